# Initial kernel scaffold; baseline (speedup 1.0000x reference)
import jax
import jax.numpy as jnp
from jax.experimental import pallas as pl


def kernel(embs, seq_lengths, table):
    raise NotImplementedError("write your pallas kernel here")


# speedup vs baseline: 6.6737x; 6.6737x over previous
import jax
import jax.numpy as jnp
from jax import lax
from jax.experimental import pallas as pl
from jax.experimental.pallas import tpu as pltpu


def _body(thresh_ref, embs_ref, tbl_ref, out_ref):
    bb, ld = embs_ref.shape
    col = lax.broadcasted_iota(jnp.int32, (bb, ld), 1)
    mask = col < thresh_ref[...]
    tbl = tbl_ref[...]
    out_ref[...] = embs_ref[...] + jnp.where(mask, tbl, 0.0)


def kernel(embs, seq_lengths, table):
    B, L, D = embs.shape
    LD = L * D
    embs2 = embs.reshape(B, LD)
    tbl = table[1:L + 1].reshape(1, LD)
    thresh = (seq_lengths.astype(jnp.int32) * D).reshape(B, 1)

    BB = 128
    grid = (B // BB,)
    out = pl.pallas_call(
        _body,
        grid=grid,
        in_specs=[
            pl.BlockSpec((BB, 1), lambda i: (i, 0)),
            pl.BlockSpec((BB, LD), lambda i: (i, 0)),
            pl.BlockSpec((1, LD), lambda i: (0, 0)),
        ],
        out_specs=pl.BlockSpec((BB, LD), lambda i: (i, 0)),
        out_shape=jax.ShapeDtypeStruct((B, LD), jnp.float32),
    )(thresh, embs2, tbl)
    return out.reshape(B, L, D)
